# baseline (speedup 1.0000x reference)
import functools

import jax
import jax.numpy as jnp
from jax import lax
from jax.experimental import pallas as pl
from jax.experimental.pallas import tpu as pltpu
from jax.experimental.pallas import tpu_sc as plsc

B = 16384
D = 64
T = 1000

NC = 2
NS = 16
L = 16
NW = NC * NS
RW = B // NW
KI = RW // 128

_mesh = plsc.VectorSubcoreMesh(core_axis_name="c", subcore_axis_name="s")


@functools.partial(
    pl.kernel,
    out_type=[jax.ShapeDtypeStruct((B,), jnp.float32),
              jax.ShapeDtypeStruct((B,), jnp.float32)],
    mesh=_mesh,
    scratch_types=[
        pltpu.VMEM((RW,), jnp.int32),
        pltpu.VMEM((KI, 128), jnp.int32),
        pltpu.VMEM((RW,), jnp.float32),
        pltpu.VMEM((RW,), jnp.float32),
        pltpu.SemaphoreType.DMA,
        pltpu.SemaphoreType.DMA,
    ],
)
def _gather_sc(t_hbm, tab1_hbm, tab2_hbm, c1_hbm, c2_hbm,
               t_v, idx_v, c1_v, c2_v, sem_g, sem_out):
    wid = lax.axis_index("s") * NC + lax.axis_index("c")
    row0 = wid * RW

    pltpu.sync_copy(t_hbm.at[pl.ds(row0, RW)], t_v)

    for jc in range(RW // L):
        tv = t_v[pl.ds(jc * L, L)]
        lane = (jc % 8) * L + lax.iota(jnp.int32, L)
        band = wid * KI + jc // 8
        idx = ((tv >> 3) * (B * 8) + ((tv & 7) << 7)
               + (band * 1024 + lane))
        idx_v[jc // 8, pl.ds((jc % 8) * L, L)] = idx

    gathers = []
    for k in range(KI):
        gathers.append(
            pltpu.async_copy(tab1_hbm.at[idx_v.at[k]],
                             c1_v.at[pl.ds(k * 128, 128)], sem_g))
        gathers.append(
            pltpu.async_copy(tab2_hbm.at[idx_v.at[k]],
                             c2_v.at[pl.ds(k * 128, 128)], sem_g))
    for g in gathers:
        g.wait()

    outs = [pltpu.async_copy(c1_v, c1_hbm.at[pl.ds(row0, RW)], sem_out),
            pltpu.async_copy(c2_v, c2_hbm.at[pl.ds(row0, RW)], sem_out)]
    for cp in outs:
        cp.wait()


def _phys_flat(a, band):
    n, c = a.shape
    v = a.reshape(n // 128, 128, c // band, band)
    return v.transpose(2, 0, 3, 1).reshape(n * c)


BN = 16384


def _scale_add_tc(x_ref, n_ref, c1_ref, c2_ref, o_ref):
    o_ref[...] = c1_ref[...] * x_ref[...] + c2_ref[...] * n_ref[...]


_tc_call = pl.pallas_call(
    _scale_add_tc,
    grid=(B // BN,),
    in_specs=[pl.BlockSpec((D, BN), lambda i: (0, i)),
              pl.BlockSpec((D, BN), lambda i: (0, i)),
              pl.BlockSpec((1, BN), lambda i: (0, i)),
              pl.BlockSpec((1, BN), lambda i: (0, i))],
    out_specs=pl.BlockSpec((D, BN), lambda i: (0, i)),
    out_shape=jax.ShapeDtypeStruct((D, B), jnp.float32),
)


def kernel(x_start, t, noise, sqrt_alphas_cumprod, sqrt_one_minus_alphas_cumprod):
    c1, c2 = _gather_sc(
        t.astype(jnp.int32),
        _phys_flat(sqrt_alphas_cumprod, 8),
        _phys_flat(sqrt_one_minus_alphas_cumprod, 8),
    )
    out_t = _tc_call(x_start.T, noise.T, c1.reshape(1, B), c2.reshape(1, B))
    return out_t.T


# speedup vs baseline: 1.0408x; 1.0408x over previous
import functools

import jax
import jax.numpy as jnp
from jax import lax
from jax.experimental import pallas as pl
from jax.experimental.pallas import tpu as pltpu
from jax.experimental.pallas import tpu_sc as plsc

B = 16384
D = 64
T = 1000

NC = 2
NS = 16
L = 16
NW = NC * NS
RW = B // NW
KI = RW // 128

_mesh = plsc.VectorSubcoreMesh(core_axis_name="c", subcore_axis_name="s")


@functools.partial(
    pl.kernel,
    out_type=[jax.ShapeDtypeStruct((B,), jnp.float32),
              jax.ShapeDtypeStruct((B,), jnp.float32)],
    mesh=_mesh,
    scratch_types=[
        pltpu.VMEM((RW,), jnp.int32),
        pltpu.VMEM((KI, 128), jnp.int32),
        pltpu.VMEM((RW,), jnp.float32),
        pltpu.VMEM((RW,), jnp.float32),
        pltpu.SemaphoreType.DMA,
        pltpu.SemaphoreType.DMA,
    ],
)
def _gather_sc(t_hbm, tab1_hbm, tab2_hbm, c1_hbm, c2_hbm,
               t_v, idx_v, c1_v, c2_v, sem_g, sem_out):
    wid = lax.axis_index("s") * NC + lax.axis_index("c")
    row0 = wid * RW

    pltpu.sync_copy(t_hbm.at[pl.ds(row0, RW)], t_v)

    for jc in range(RW // L):
        tv = t_v[pl.ds(jc * L, L)]
        lane = (jc % 8) * L + lax.iota(jnp.int32, L)
        band = wid * KI + jc // 8
        idx = ((tv >> 3) * (B * 8) + ((tv & 7) << 7)
               + (band * 1024 + lane))
        idx_v[jc // 8, pl.ds((jc % 8) * L, L)] = idx

    gathers = []
    for k in range(KI):
        gathers.append(
            pltpu.async_copy(tab1_hbm.at[idx_v.at[k]],
                             c1_v.at[pl.ds(k * 128, 128)], sem_g))
        gathers.append(
            pltpu.async_copy(tab2_hbm.at[idx_v.at[k]],
                             c2_v.at[pl.ds(k * 128, 128)], sem_g))
    for g in gathers:
        g.wait()

    outs = [pltpu.async_copy(c1_v, c1_hbm.at[pl.ds(row0, RW)], sem_out),
            pltpu.async_copy(c2_v, c2_hbm.at[pl.ds(row0, RW)], sem_out)]
    for cp in outs:
        cp.wait()


def _phys_flat(a, band):
    n, c = a.shape
    v = a.reshape(n // 128, 128, c // band, band)
    return v.transpose(2, 0, 3, 1).reshape(n * c)


BN = 8192


def _scale_add_tc(x_ref, n_ref, c1_ref, c2_ref, o_ref):
    o_ref[...] = c1_ref[...] * x_ref[...] + c2_ref[...] * n_ref[...]


_tc_call = pl.pallas_call(
    _scale_add_tc,
    grid=(B // BN,),
    in_specs=[pl.BlockSpec((D, BN), lambda i: (0, i)),
              pl.BlockSpec((D, BN), lambda i: (0, i)),
              pl.BlockSpec((1, BN), lambda i: (0, i)),
              pl.BlockSpec((1, BN), lambda i: (0, i))],
    out_specs=pl.BlockSpec((D, BN), lambda i: (0, i)),
    out_shape=jax.ShapeDtypeStruct((D, B), jnp.float32),
)


def kernel(x_start, t, noise, sqrt_alphas_cumprod, sqrt_one_minus_alphas_cumprod):
    c1, c2 = _gather_sc(
        t.astype(jnp.int32),
        _phys_flat(sqrt_alphas_cumprod, 8),
        _phys_flat(sqrt_one_minus_alphas_cumprod, 8),
    )
    out_t = _tc_call(x_start.T, noise.T, c1.reshape(1, B), c2.reshape(1, B))
    return out_t.T
